# baseline (speedup 1.0000x reference)
import functools
import math

import jax
import jax.numpy as jnp
from jax import lax
from jax.experimental import pallas as pl
from jax.experimental.pallas import tpu as pltpu
from jax.experimental.pallas import tpu_sc as plsc

_B, _N, _K, _EMB, _HEADS = 8, 1024, 20, 1024, 4
_EPS = 1e-5
_RB = 256
_NEG = float("-inf")
_SLOPE = 0.2


def _leaky(v):
    return jnp.where(v >= 0, v, _SLOPE * v)


def _attn_conv_body(xf_ref, xb_ref, wq_ref, wf_ref, g_ref, b_ref, out_ref):
    X = xf_ref[0]
    Xb = xb_ref[0]
    c = X.shape[0]

    G = jax.lax.dot_general(Xb, X, (((0,), (0,)), ((), ())),
                            preferred_element_type=jnp.float32)
    xx_row = jnp.sum(X * X, axis=0, keepdims=True)
    ones_c = jnp.ones((c, 1), jnp.float32)
    xx_col = jax.lax.dot_general(Xb * Xb, ones_c, (((0,), (0,)), ((), ())),
                                 preferred_element_type=jnp.float32)
    pd = 2.0 * G - xx_col - xx_row

    t = jnp.max(pd, axis=1, keepdims=True)
    selfmax = t
    for _ in range(_K):
        t = jnp.max(jnp.where(pd < t, pd, _NEG), axis=1, keepdims=True)
    member = jnp.logical_and(pd >= t, pd < selfmax)

    m_all = jnp.max(G, axis=1, keepdims=True)
    e = jnp.where(member, jnp.exp(G - m_all), 0.0)
    Zrow = jax.lax.dot_general(jnp.ones((1, pd.shape[1]), jnp.float32), e,
                               (((1,), (1,)), ((), ())),
                               preferred_element_type=jnp.float32)

    FT = jax.lax.dot_general(X, e, (((1,), (1,)), ((), ())),
                             preferred_element_type=jnp.float32)
    val = FT / Zrow - Xb

    Y = jax.lax.dot_general(wq_ref[...], Xb, (((1,), (0,)), ((), ())),
                            preferred_element_type=jnp.float32)
    Y = Y + jax.lax.dot_general(wf_ref[...], val, (((1,), (0,)), ((), ())),
                                preferred_element_type=jnp.float32)
    Y = Y / jnp.sqrt(jnp.float32(1.0 + _EPS))
    Y = Y * g_ref[...] + b_ref[...]
    out_ref[0] = _leaky(Y)


def _attn_conv(x, wq, wf, g, b):
    Bn, c, n = x.shape
    oc = wq.shape[0]
    grid = (Bn, n // _RB)
    return pl.pallas_call(
        _attn_conv_body,
        grid=grid,
        in_specs=[
            pl.BlockSpec((1, c, n), lambda bi, r: (bi, 0, 0)),
            pl.BlockSpec((1, c, _RB), lambda bi, r: (bi, 0, r)),
            pl.BlockSpec((oc, c), lambda bi, r: (0, 0)),
            pl.BlockSpec((oc, c), lambda bi, r: (0, 0)),
            pl.BlockSpec((oc, 1), lambda bi, r: (0, 0)),
            pl.BlockSpec((oc, 1), lambda bi, r: (0, 0)),
        ],
        out_specs=pl.BlockSpec((1, oc, _RB), lambda bi, r: (bi, 0, r)),
        out_shape=jax.ShapeDtypeStruct((Bn, oc, n), jnp.float32),
        compiler_params=pltpu.CompilerParams(
            dimension_semantics=("parallel", "parallel")),
    )(x, x, wq, wf, g, b)


_NC, _NS = 2, 16
_NW = _NC * _NS
_RPW = (_B * _N) // _NW
_RCH = 32


def _pd_body(xf_ref, xb_ref, pd_ref):
    X = xf_ref[0]
    Xb = xb_ref[0]
    c = X.shape[0]
    G = jax.lax.dot_general(Xb, X, (((0,), (0,)), ((), ())),
                            preferred_element_type=jnp.float32)
    xx_row = jnp.sum(X * X, axis=0, keepdims=True)
    ones_c = jnp.ones((c, 1), jnp.float32)
    xx_col = jax.lax.dot_general(Xb * Xb, ones_c, (((0,), (0,)), ((), ())),
                                 preferred_element_type=jnp.float32)
    pd_ref[0] = 2.0 * G - xx_col - xx_row


def _pd_call(x):
    Bn, c, n = x.shape
    return pl.pallas_call(
        _pd_body,
        grid=(Bn, n // _RB),
        in_specs=[
            pl.BlockSpec((1, c, n), lambda bi, r: (bi, 0, 0)),
            pl.BlockSpec((1, c, _RB), lambda bi, r: (bi, 0, r)),
        ],
        out_specs=pl.BlockSpec((1, _RB, n), lambda bi, r: (bi, r, 0)),
        out_shape=jax.ShapeDtypeStruct((Bn, n, n), jnp.float32),
        compiler_params=pltpu.CompilerParams(
            dimension_semantics=("parallel", "parallel")),
    )(x, x)


def _sc_topk_body(rpw, pd_hbm, out_hbm, rows_v, srt_v):
    wid = lax.axis_index("s") * _NC + lax.axis_index("c")
    base = wid * rpw
    nchunk = _N // 16

    def chunk_body(ci, carry):
        rbase = base + ci * _RCH
        pltpu.sync_copy(pd_hbm.at[pl.ds(rbase, _RCH)], rows_v)

        lanes = lax.iota(jnp.int32, 16)

        def xmax(v):
            for s in (8, 4, 2, 1):
                v = jnp.maximum(v, v[lanes ^ s])
            return v

        _NM = 5

        def pair_body(p, carry2):
            rr = (2 * p, 2 * p + 1)

            ms = []
            for r in rr:
                m = [rows_v[r, pl.ds(0, 16)]]
                m += [jnp.full((16,), _NEG, jnp.float32)] * (_NM - 1)
                ms.append(m)
            for i in range(1, nchunk):
                for q, r in enumerate(rr):
                    v = rows_v[r, pl.ds(i * 16, 16)]
                    m = ms[q]
                    for j in range(_NM - 1):
                        hi = jnp.maximum(m[j], v)
                        v = jnp.minimum(m[j], v)
                        m[j] = hi
                    m[_NM - 1] = jnp.maximum(m[_NM - 1], v)
            selfs = [xmax(ms[0][0]), xmax(ms[1][0])]

            def cand_chase(_, ts):
                outs = []
                for q in range(2):
                    a = jnp.full((16,), _NEG, jnp.float32)
                    for j in range(_NM):
                        a = jnp.maximum(
                            a, jnp.where(ms[q][j] < ts[q], ms[q][j], _NEG))
                    outs.append(xmax(a))
                return tuple(outs)

            t_hats = lax.fori_loop(0, _K, cand_chase, tuple(selfs))

            one = jnp.ones((16,), jnp.int32)
            zero = jnp.zeros((16,), jnp.int32)
            cnts = [zero, zero]
            for i in range(nchunk):
                for q, r in enumerate(rr):
                    c = rows_v[r, pl.ds(i * 16, 16)]
                    cnts[q] = cnts[q] + jnp.where(c >= t_hats[q], one, zero)
            for s in (8, 4, 2, 1):
                cnts = [cnt + cnt[lanes ^ s] for cnt in cnts]

            for q, r in enumerate(rr):
                def full_chase(_, t, r=r):
                    a = jnp.full((16,), _NEG, jnp.float32)
                    for i in range(nchunk):
                        c = rows_v[r, pl.ds(i * 16, 16)]
                        a = jnp.maximum(a, jnp.where(c < t, c, _NEG))
                    return xmax(a)

                t21_s = lax.cond(
                    cnts[q][0] == _K + 1,
                    lambda q=q: t_hats[q][0],
                    lambda q=q, fc=full_chase: lax.fori_loop(
                        0, _K, fc, selfs[q])[0])
                t21 = jnp.full((16,), t21_s, jnp.float32)
                out = jnp.where(lanes == 0, selfs[q],
                                jnp.where(lanes == 1, t21, 0.0))
                srt_v[r, pl.ds(0, 16)] = out
            return carry2

        lax.fori_loop(0, _RCH // 2, pair_body, 0)
        pltpu.sync_copy(srt_v, out_hbm.at[pl.ds(rbase, _RCH)])
        return carry

    lax.fori_loop(0, rpw // _RCH, chunk_body, 0)


@functools.lru_cache(maxsize=None)
def _sc_topk(nrows):
    return pl.kernel(
        functools.partial(_sc_topk_body, nrows // _NW),
        out_type=jax.ShapeDtypeStruct((nrows, 16), jnp.float32),
        mesh=plsc.VectorSubcoreMesh(core_axis_name="c", subcore_axis_name="s"),
        scratch_types=[
            pltpu.VMEM((_RCH, _N), jnp.float32),
            pltpu.VMEM((_RCH, 16), jnp.float32),
        ],
    )


def _post_body(xf_ref, xb_ref, thr_ref, wq_ref, wf_ref, g_ref, b_ref,
               out_ref):
    X = xf_ref[0]
    Xb = xb_ref[0]
    c = X.shape[0]
    G = jax.lax.dot_general(Xb, X, (((0,), (0,)), ((), ())),
                            preferred_element_type=jnp.float32)
    xx_row = jnp.sum(X * X, axis=0, keepdims=True)
    ones_c = jnp.ones((c, 1), jnp.float32)
    xx_col = jax.lax.dot_general(Xb * Xb, ones_c, (((0,), (0,)), ((), ())),
                                 preferred_element_type=jnp.float32)
    pd = 2.0 * G - xx_col - xx_row

    thr = thr_ref[...]
    selfmax = thr[:, 0:1]
    t21 = thr[:, 1:2]
    member = jnp.logical_and(pd >= t21, pd < selfmax)

    m_all = jnp.max(G, axis=1, keepdims=True)
    e = jnp.where(member, jnp.exp(G - m_all), 0.0)
    Zrow = jax.lax.dot_general(jnp.ones((1, pd.shape[1]), jnp.float32), e,
                               (((1,), (1,)), ((), ())),
                               preferred_element_type=jnp.float32)
    FT = jax.lax.dot_general(X, e, (((1,), (1,)), ((), ())),
                             preferred_element_type=jnp.float32)
    val = FT / Zrow - Xb

    Y = jax.lax.dot_general(wq_ref[...], Xb, (((1,), (0,)), ((), ())),
                            preferred_element_type=jnp.float32)
    Y = Y + jax.lax.dot_general(wf_ref[...], val, (((1,), (0,)), ((), ())),
                                preferred_element_type=jnp.float32)
    Y = Y / jnp.sqrt(jnp.float32(1.0 + _EPS))
    Y = Y * g_ref[...] + b_ref[...]
    out_ref[0] = _leaky(Y)


def _post_call(x, thr, wq, wf, g, b):
    Bn, c, n = x.shape
    oc = wq.shape[0]
    nrb = n // _RB
    return pl.pallas_call(
        _post_body,
        grid=(Bn, nrb),
        in_specs=[
            pl.BlockSpec((1, c, n), lambda bi, r: (bi, 0, 0)),
            pl.BlockSpec((1, c, _RB), lambda bi, r: (bi, 0, r)),
            pl.BlockSpec((_RB, 16), lambda bi, r: (bi * 4 + r, 0)),
            pl.BlockSpec((oc, c), lambda bi, r: (0, 0)),
            pl.BlockSpec((oc, c), lambda bi, r: (0, 0)),
            pl.BlockSpec((oc, 1), lambda bi, r: (0, 0)),
            pl.BlockSpec((oc, 1), lambda bi, r: (0, 0)),
        ],
        out_specs=pl.BlockSpec((1, oc, _RB), lambda bi, r: (bi, 0, r)),
        out_shape=jax.ShapeDtypeStruct((Bn, oc, n), jnp.float32),
        compiler_params=pltpu.CompilerParams(
            dimension_semantics=("parallel", "parallel")),
    )(x, x, thr, wq, wf, g, b)


_SCB = 4


def _attn_conv_sc(x, wq, wf, g, b):
    Bn = x.shape[0]
    pd = _pd_call(x)
    thr = _sc_topk(Bn * _N)(pd.reshape(Bn * _N, _N))
    return _post_call(x, thr, wq, wf, g, b)


def _attn_conv_mixed(x, wq, wf, g, b):
    xs = x[:_SCB]
    pd = _pd_call(xs)
    thr = _sc_topk(_SCB * _N)(pd.reshape(_SCB * _N, _N))
    yt = _attn_conv(x[_SCB:], wq, wf, g, b)
    ys = _post_call(xs, thr, wq, wf, g, b)
    return jnp.concatenate([ys, yt], axis=0)


def _head_body(x1_ref, x2_ref, x3_ref, x4_ref, w5_ref, g5_ref, b5_ref,
               ws_ref, ap_ref):
    W5 = w5_ref[...]
    h = jax.lax.dot_general(W5[:, 0:64], x1_ref[0], (((1,), (0,)), ((), ())),
                            preferred_element_type=jnp.float32)
    h = h + jax.lax.dot_general(W5[:, 64:128], x2_ref[0],
                                (((1,), (0,)), ((), ())),
                                preferred_element_type=jnp.float32)
    h = h + jax.lax.dot_general(W5[:, 128:256], x3_ref[0],
                                (((1,), (0,)), ((), ())),
                                preferred_element_type=jnp.float32)
    h = h + jax.lax.dot_general(W5[:, 256:512], x4_ref[0],
                                (((1,), (0,)), ((), ())),
                                preferred_element_type=jnp.float32)
    h = h / jnp.sqrt(jnp.float32(1.0 + _EPS))
    h = _leaky(h * g5_ref[...] + b5_ref[...])
    S = _leaky(jax.lax.dot_general(ws_ref[...], h, (((1,), (0,)), ((), ())),
                                   preferred_element_type=jnp.float32))
    ap_ref[0] = jax.lax.dot_general(S, h, (((1,), (1,)), ((), ())),
                                    preferred_element_type=jnp.float32)


def _head(x1, x2, x3, x4, W5, g5, b5, Ws):
    return pl.pallas_call(
        _head_body,
        grid=(_B,),
        in_specs=[
            pl.BlockSpec((1, 64, _N), lambda bi: (bi, 0, 0)),
            pl.BlockSpec((1, 64, _N), lambda bi: (bi, 0, 0)),
            pl.BlockSpec((1, 128, _N), lambda bi: (bi, 0, 0)),
            pl.BlockSpec((1, 256, _N), lambda bi: (bi, 0, 0)),
            pl.BlockSpec((_EMB, 512), lambda bi: (0, 0)),
            pl.BlockSpec((_EMB, 1), lambda bi: (0, 0)),
            pl.BlockSpec((_EMB, 1), lambda bi: (0, 0)),
            pl.BlockSpec((_HEADS, _EMB), lambda bi: (0, 0)),
        ],
        out_specs=pl.BlockSpec((1, _HEADS, _EMB), lambda bi: (bi, 0, 0)),
        out_shape=jax.ShapeDtypeStruct((_B, _HEADS, _EMB), jnp.float32),
        compiler_params=pltpu.CompilerParams(
            dimension_semantics=("parallel",)),
    )(x1, x2, x3, x4, W5, g5, b5, Ws)


def _mlp_body(ap_ref, lng_ref, lnb_ref, wl1_ref, bl1_ref, g6_ref, b6_ref,
              wl2_ref, bl2_ref, g7_ref, b7_ref, wl3_ref, bl3_ref, out_ref):
    ap = ap_ref[...]
    mu = jnp.mean(ap, axis=1, keepdims=True)
    d = ap - mu
    var = jnp.mean(d * d, axis=1, keepdims=True)
    ap = d / jnp.sqrt(var + _EPS) * lng_ref[...] + lnb_ref[...]
    ap = _leaky(ap)
    sq = jnp.sqrt(jnp.float32(1.0 + _EPS))
    y = jax.lax.dot_general(ap, wl1_ref[...], (((1,), (1,)), ((), ())),
                            preferred_element_type=jnp.float32) + bl1_ref[...]
    y = _leaky(y / sq * g6_ref[...] + b6_ref[...])
    y = jax.lax.dot_general(y, wl2_ref[...], (((1,), (1,)), ((), ())),
                            preferred_element_type=jnp.float32) + bl2_ref[...]
    y = _leaky(y / sq * g7_ref[...] + b7_ref[...])
    out_ref[...] = jax.lax.dot_general(
        y, wl3_ref[...], (((1,), (1,)), ((), ())),
        preferred_element_type=jnp.float32) + bl3_ref[...]


def _mlp(ap, lng, lnb, Wl1, bl1, g6, b6, Wl2, bl2, g7, b7, Wl3, bl3):
    return pl.pallas_call(
        _mlp_body,
        out_shape=jax.ShapeDtypeStruct((_B, 40), jnp.float32),
    )(ap, lng, lnb, Wl1, bl1, g6, b6, Wl2, bl2, g7, b7, Wl3, bl3)


def kernel(x, W1, g1, b1, W2, g2, b2, W3, g3, b3, W4, g4, b4, W5, g5, b5,
           Ws, lng, lnb, Wl1, bl1, g6, b6, Wl2, bl2, g7, b7, Wl3, bl3):
    f32 = jnp.float32
    x8 = jnp.concatenate([x, jnp.zeros((_B, 5, _N), f32)], axis=1)
    Wq1 = jnp.pad(W1[:, 0:3], ((0, 0), (0, 5)))
    Wf1 = jnp.pad(W1[:, 3:6], ((0, 0), (0, 5)))

    col = lambda v: v[:, None]
    row = lambda v: v[None, :]

    x1 = _attn_conv_mixed(x8, Wq1, Wf1, col(g1), col(b1))
    x2 = _attn_conv_mixed(x1, W2[:, 0:64], W2[:, 64:128], col(g2), col(b2))
    x3 = _attn_conv_mixed(x2, W3[:, 0:64], W3[:, 64:128], col(g3), col(b3))
    x4 = _attn_conv_mixed(x3, W4[:, 0:128], W4[:, 128:256], col(g4), col(b4))

    ap = _head(x1, x2, x3, x4, W5, col(g5), col(b5), Ws)
    ap = ap.reshape(_B, _HEADS * _EMB)

    return _mlp(ap, row(lng), row(lnb), Wl1, row(bl1), row(g6), row(b6),
                Wl2, row(bl2), row(g7), row(b7), Wl3, row(bl3))


# speedup vs baseline: 1.0026x; 1.0026x over previous
import functools

import jax
import jax.numpy as jnp
from jax import lax
from jax.experimental import pallas as pl
from jax.experimental.pallas import tpu as pltpu
from jax.experimental.pallas import tpu_sc as plsc

_B, _N, _K, _EMB, _HEADS = 8, 1024, 20, 1024, 4
_EPS = 1e-5
_RB = 256
_NEG = float("-inf")
_SLOPE = 0.2


def _leaky(v):
    return jnp.where(v >= 0, v, _SLOPE * v)


def _attn_conv_body(xf_ref, xb_ref, wq_ref, wf_ref, g_ref, b_ref, out_ref):
    X = xf_ref[0]
    Xb = xb_ref[0]
    c = X.shape[0]

    G = jax.lax.dot_general(Xb, X, (((0,), (0,)), ((), ())),
                            preferred_element_type=jnp.float32)
    xx_row = jnp.sum(X * X, axis=0, keepdims=True)
    ones_c = jnp.ones((c, 1), jnp.float32)
    xx_col = jax.lax.dot_general(Xb * Xb, ones_c, (((0,), (0,)), ((), ())),
                                 preferred_element_type=jnp.float32)
    pd = 2.0 * G - xx_col - xx_row

    t = jnp.max(pd, axis=1, keepdims=True)
    selfmax = t
    for _ in range(_K):
        t = jnp.max(jnp.where(pd < t, pd, _NEG), axis=1, keepdims=True)
    member = jnp.logical_and(pd >= t, pd < selfmax)

    m_all = jnp.max(G, axis=1, keepdims=True)
    e = jnp.where(member, jnp.exp(G - m_all), 0.0)
    Zrow = jax.lax.dot_general(jnp.ones((1, pd.shape[1]), jnp.float32), e,
                               (((1,), (1,)), ((), ())),
                               preferred_element_type=jnp.float32)

    FT = jax.lax.dot_general(X, e, (((1,), (1,)), ((), ())),
                             preferred_element_type=jnp.float32)
    val = FT / Zrow - Xb

    Y = jax.lax.dot_general(wq_ref[...], Xb, (((1,), (0,)), ((), ())),
                            preferred_element_type=jnp.float32)
    Y = Y + jax.lax.dot_general(wf_ref[...], val, (((1,), (0,)), ((), ())),
                                preferred_element_type=jnp.float32)
    Y = Y / jnp.sqrt(jnp.float32(1.0 + _EPS))
    Y = Y * g_ref[...] + b_ref[...]
    out_ref[0] = _leaky(Y)


def _attn_conv(x, wq, wf, g, b):
    Bn, c, n = x.shape
    oc = wq.shape[0]
    grid = (Bn, n // _RB)
    return pl.pallas_call(
        _attn_conv_body,
        grid=grid,
        in_specs=[
            pl.BlockSpec((1, c, n), lambda bi, r: (bi, 0, 0)),
            pl.BlockSpec((1, c, _RB), lambda bi, r: (bi, 0, r)),
            pl.BlockSpec((oc, c), lambda bi, r: (0, 0)),
            pl.BlockSpec((oc, c), lambda bi, r: (0, 0)),
            pl.BlockSpec((oc, 1), lambda bi, r: (0, 0)),
            pl.BlockSpec((oc, 1), lambda bi, r: (0, 0)),
        ],
        out_specs=pl.BlockSpec((1, oc, _RB), lambda bi, r: (bi, 0, r)),
        out_shape=jax.ShapeDtypeStruct((Bn, oc, n), jnp.float32),
        compiler_params=pltpu.CompilerParams(
            dimension_semantics=("parallel", "parallel")),
    )(x, x, wq, wf, g, b)


_NC, _NS = 2, 16
_NW = _NC * _NS
_RCH = 32


def _pd_body(xf_ref, xb_ref, pd_ref):
    X = xf_ref[0]
    Xb = xb_ref[0]
    c = X.shape[0]
    G = jax.lax.dot_general(Xb, X, (((0,), (0,)), ((), ())),
                            preferred_element_type=jnp.float32)
    xx_row = jnp.sum(X * X, axis=0, keepdims=True)
    ones_c = jnp.ones((c, 1), jnp.float32)
    xx_col = jax.lax.dot_general(Xb * Xb, ones_c, (((0,), (0,)), ((), ())),
                                 preferred_element_type=jnp.float32)
    pd_ref[0] = 2.0 * G - xx_col - xx_row


def _pd_call(x):
    Bn, c, n = x.shape
    return pl.pallas_call(
        _pd_body,
        grid=(Bn, n // _RB),
        in_specs=[
            pl.BlockSpec((1, c, n), lambda bi, r: (bi, 0, 0)),
            pl.BlockSpec((1, c, _RB), lambda bi, r: (bi, 0, r)),
        ],
        out_specs=pl.BlockSpec((1, _RB, n), lambda bi, r: (bi, r, 0)),
        out_shape=jax.ShapeDtypeStruct((Bn, n, n), jnp.float32),
        compiler_params=pltpu.CompilerParams(
            dimension_semantics=("parallel", "parallel")),
    )(x, x)


def _sc_topk_body(rpw, pd_hbm, out_hbm, rows_v, srt_v):
    wid = lax.axis_index("s") * _NC + lax.axis_index("c")
    base = wid * rpw
    nchunk = _N // 16

    def chunk_body(ci, carry):
        rbase = base + ci * _RCH
        pltpu.sync_copy(pd_hbm.at[pl.ds(rbase, _RCH)], rows_v)

        lanes = lax.iota(jnp.int32, 16)

        def xmax(v):
            for s in (8, 4, 2, 1):
                v = jnp.maximum(v, v[lanes ^ s])
            return v

        _NM = 5

        def pair_body(p, carry2):
            rr = (2 * p, 2 * p + 1)

            ms = []
            for r in rr:
                m = [rows_v[r, pl.ds(0, 16)]]
                m += [jnp.full((16,), _NEG, jnp.float32)] * (_NM - 1)
                ms.append(m)
            for i in range(1, nchunk):
                for q, r in enumerate(rr):
                    v = rows_v[r, pl.ds(i * 16, 16)]
                    m = ms[q]
                    for j in range(_NM - 1):
                        hi = jnp.maximum(m[j], v)
                        v = jnp.minimum(m[j], v)
                        m[j] = hi
                    m[_NM - 1] = jnp.maximum(m[_NM - 1], v)
            selfs = [xmax(ms[0][0]), xmax(ms[1][0])]

            def cand_chase(_, ts):
                outs = []
                for q in range(2):
                    a = jnp.full((16,), _NEG, jnp.float32)
                    for j in range(_NM):
                        a = jnp.maximum(
                            a, jnp.where(ms[q][j] < ts[q], ms[q][j], _NEG))
                    outs.append(xmax(a))
                return tuple(outs)

            t_hats = lax.fori_loop(0, _K, cand_chase, tuple(selfs))

            one = jnp.ones((16,), jnp.int32)
            zero = jnp.zeros((16,), jnp.int32)
            cnts = [zero, zero]
            for i in range(nchunk):
                for q, r in enumerate(rr):
                    c = rows_v[r, pl.ds(i * 16, 16)]
                    cnts[q] = cnts[q] + jnp.where(c >= t_hats[q], one, zero)
            for s in (8, 4, 2, 1):
                cnts = [cnt + cnt[lanes ^ s] for cnt in cnts]

            for q, r in enumerate(rr):
                def full_chase(_, t, r=r):
                    a = jnp.full((16,), _NEG, jnp.float32)
                    for i in range(nchunk):
                        c = rows_v[r, pl.ds(i * 16, 16)]
                        a = jnp.maximum(a, jnp.where(c < t, c, _NEG))
                    return xmax(a)

                t21_s = lax.cond(
                    cnts[q][0] == _K + 1,
                    lambda q=q: t_hats[q][0],
                    lambda q=q, fc=full_chase: lax.fori_loop(
                        0, _K, fc, selfs[q])[0])
                t21 = jnp.full((16,), t21_s, jnp.float32)
                out = jnp.where(lanes == 0, selfs[q],
                                jnp.where(lanes == 1, t21, 0.0))
                srt_v[r, pl.ds(0, 16)] = out
            return carry2

        lax.fori_loop(0, _RCH // 2, pair_body, 0)
        pltpu.sync_copy(srt_v, out_hbm.at[pl.ds(rbase, _RCH)])
        return carry

    lax.fori_loop(0, rpw // _RCH, chunk_body, 0)


@functools.lru_cache(maxsize=None)
def _sc_topk(nrows):
    return pl.kernel(
        functools.partial(_sc_topk_body, nrows // _NW),
        out_type=jax.ShapeDtypeStruct((nrows, 16), jnp.float32),
        mesh=plsc.VectorSubcoreMesh(core_axis_name="c", subcore_axis_name="s"),
        scratch_types=[
            pltpu.VMEM((_RCH, _N), jnp.float32),
            pltpu.VMEM((_RCH, 16), jnp.float32),
        ],
    )


def _post_body(xf_ref, xb_ref, thr_ref, wq_ref, wf_ref, g_ref, b_ref,
               out_ref):
    X = xf_ref[0]
    Xb = xb_ref[0]
    c = X.shape[0]
    G = jax.lax.dot_general(Xb, X, (((0,), (0,)), ((), ())),
                            preferred_element_type=jnp.float32)
    xx_row = jnp.sum(X * X, axis=0, keepdims=True)
    ones_c = jnp.ones((c, 1), jnp.float32)
    xx_col = jax.lax.dot_general(Xb * Xb, ones_c, (((0,), (0,)), ((), ())),
                                 preferred_element_type=jnp.float32)
    pd = 2.0 * G - xx_col - xx_row

    thr = thr_ref[...]
    selfmax = thr[:, 0:1]
    t21 = thr[:, 1:2]
    member = jnp.logical_and(pd >= t21, pd < selfmax)

    m_all = jnp.max(G, axis=1, keepdims=True)
    e = jnp.where(member, jnp.exp(G - m_all), 0.0)
    Zrow = jax.lax.dot_general(jnp.ones((1, pd.shape[1]), jnp.float32), e,
                               (((1,), (1,)), ((), ())),
                               preferred_element_type=jnp.float32)
    FT = jax.lax.dot_general(X, e, (((1,), (1,)), ((), ())),
                             preferred_element_type=jnp.float32)
    val = FT / Zrow - Xb

    Y = jax.lax.dot_general(wq_ref[...], Xb, (((1,), (0,)), ((), ())),
                            preferred_element_type=jnp.float32)
    Y = Y + jax.lax.dot_general(wf_ref[...], val, (((1,), (0,)), ((), ())),
                                preferred_element_type=jnp.float32)
    Y = Y / jnp.sqrt(jnp.float32(1.0 + _EPS))
    Y = Y * g_ref[...] + b_ref[...]
    out_ref[0] = _leaky(Y)


def _post_call(x, thr, wq, wf, g, b):
    Bn, c, n = x.shape
    oc = wq.shape[0]
    nrb = n // _RB
    return pl.pallas_call(
        _post_body,
        grid=(Bn, nrb),
        in_specs=[
            pl.BlockSpec((1, c, n), lambda bi, r: (bi, 0, 0)),
            pl.BlockSpec((1, c, _RB), lambda bi, r: (bi, 0, r)),
            pl.BlockSpec((_RB, 16), lambda bi, r: (bi * 4 + r, 0)),
            pl.BlockSpec((oc, c), lambda bi, r: (0, 0)),
            pl.BlockSpec((oc, c), lambda bi, r: (0, 0)),
            pl.BlockSpec((oc, 1), lambda bi, r: (0, 0)),
            pl.BlockSpec((oc, 1), lambda bi, r: (0, 0)),
        ],
        out_specs=pl.BlockSpec((1, oc, _RB), lambda bi, r: (bi, 0, r)),
        out_shape=jax.ShapeDtypeStruct((Bn, oc, n), jnp.float32),
        compiler_params=pltpu.CompilerParams(
            dimension_semantics=("parallel", "parallel")),
    )(x, x, thr, wq, wf, g, b)


_SCB = 4


def _attn_conv_mixed(x, wq, wf, g, b):
    xs = x[:_SCB]
    pd = _pd_call(xs)
    thr = _sc_topk(_SCB * _N)(pd.reshape(_SCB * _N, _N))
    yt = _attn_conv(x[_SCB:], wq, wf, g, b)
    ys = _post_call(xs, thr, wq, wf, g, b)
    return jnp.concatenate([ys, yt], axis=0)


def _head_body(x1_ref, x2_ref, x3_ref, x4_ref, w5_ref, g5_ref, b5_ref,
               ws_ref, ap_ref):
    W5 = w5_ref[...]
    h = jax.lax.dot_general(W5[:, 0:64], x1_ref[0], (((1,), (0,)), ((), ())),
                            preferred_element_type=jnp.float32)
    h = h + jax.lax.dot_general(W5[:, 64:128], x2_ref[0],
                                (((1,), (0,)), ((), ())),
                                preferred_element_type=jnp.float32)
    h = h + jax.lax.dot_general(W5[:, 128:256], x3_ref[0],
                                (((1,), (0,)), ((), ())),
                                preferred_element_type=jnp.float32)
    h = h + jax.lax.dot_general(W5[:, 256:512], x4_ref[0],
                                (((1,), (0,)), ((), ())),
                                preferred_element_type=jnp.float32)
    h = h / jnp.sqrt(jnp.float32(1.0 + _EPS))
    h = _leaky(h * g5_ref[...] + b5_ref[...])
    S = _leaky(jax.lax.dot_general(ws_ref[...], h, (((1,), (0,)), ((), ())),
                                   preferred_element_type=jnp.float32))
    ap_ref[0] = jax.lax.dot_general(S, h, (((1,), (1,)), ((), ())),
                                    preferred_element_type=jnp.float32)


def _head(x1, x2, x3, x4, W5, g5, b5, Ws):
    return pl.pallas_call(
        _head_body,
        grid=(_B,),
        in_specs=[
            pl.BlockSpec((1, 64, _N), lambda bi: (bi, 0, 0)),
            pl.BlockSpec((1, 64, _N), lambda bi: (bi, 0, 0)),
            pl.BlockSpec((1, 128, _N), lambda bi: (bi, 0, 0)),
            pl.BlockSpec((1, 256, _N), lambda bi: (bi, 0, 0)),
            pl.BlockSpec((_EMB, 512), lambda bi: (0, 0)),
            pl.BlockSpec((_EMB, 1), lambda bi: (0, 0)),
            pl.BlockSpec((_EMB, 1), lambda bi: (0, 0)),
            pl.BlockSpec((_HEADS, _EMB), lambda bi: (0, 0)),
        ],
        out_specs=pl.BlockSpec((1, _HEADS, _EMB), lambda bi: (bi, 0, 0)),
        out_shape=jax.ShapeDtypeStruct((_B, _HEADS, _EMB), jnp.float32),
        compiler_params=pltpu.CompilerParams(
            dimension_semantics=("parallel",)),
    )(x1, x2, x3, x4, W5, g5, b5, Ws)


def _mlp_body(ap_ref, lng_ref, lnb_ref, wl1_ref, bl1_ref, g6_ref, b6_ref,
              wl2_ref, bl2_ref, g7_ref, b7_ref, wl3_ref, bl3_ref, out_ref):
    ap = ap_ref[...]
    mu = jnp.mean(ap, axis=1, keepdims=True)
    d = ap - mu
    var = jnp.mean(d * d, axis=1, keepdims=True)
    ap = d / jnp.sqrt(var + _EPS) * lng_ref[...] + lnb_ref[...]
    ap = _leaky(ap)
    sq = jnp.sqrt(jnp.float32(1.0 + _EPS))
    y = jax.lax.dot_general(ap, wl1_ref[...], (((1,), (1,)), ((), ())),
                            preferred_element_type=jnp.float32) + bl1_ref[...]
    y = _leaky(y / sq * g6_ref[...] + b6_ref[...])
    y = jax.lax.dot_general(y, wl2_ref[...], (((1,), (1,)), ((), ())),
                            preferred_element_type=jnp.float32) + bl2_ref[...]
    y = _leaky(y / sq * g7_ref[...] + b7_ref[...])
    out_ref[...] = jax.lax.dot_general(
        y, wl3_ref[...], (((1,), (1,)), ((), ())),
        preferred_element_type=jnp.float32) + bl3_ref[...]


def _mlp(ap, lng, lnb, Wl1, bl1, g6, b6, Wl2, bl2, g7, b7, Wl3, bl3):
    return pl.pallas_call(
        _mlp_body,
        out_shape=jax.ShapeDtypeStruct((_B, 40), jnp.float32),
    )(ap, lng, lnb, Wl1, bl1, g6, b6, Wl2, bl2, g7, b7, Wl3, bl3)


def kernel(x, W1, g1, b1, W2, g2, b2, W3, g3, b3, W4, g4, b4, W5, g5, b5,
           Ws, lng, lnb, Wl1, bl1, g6, b6, Wl2, bl2, g7, b7, Wl3, bl3):
    f32 = jnp.float32
    x8 = jnp.concatenate([x, jnp.zeros((_B, 5, _N), f32)], axis=1)
    Wq1 = jnp.pad(W1[:, 0:3], ((0, 0), (0, 5)))
    Wf1 = jnp.pad(W1[:, 3:6], ((0, 0), (0, 5)))

    col = lambda v: v[:, None]
    row = lambda v: v[None, :]

    x1 = _attn_conv_mixed(x8, Wq1, Wf1, col(g1), col(b1))
    x2 = _attn_conv_mixed(x1, W2[:, 0:64], W2[:, 64:128], col(g2), col(b2))
    x3 = _attn_conv_mixed(x2, W3[:, 0:64], W3[:, 64:128], col(g3), col(b3))
    x4 = _attn_conv_mixed(x3, W4[:, 0:128], W4[:, 128:256], col(g4), col(b4))

    ap = _head(x1, x2, x3, x4, W5, col(g5), col(b5), Ws)
    ap = ap.reshape(_B, _HEADS * _EMB)

    return _mlp(ap, row(lng), row(lnb), Wl1, row(bl1), row(g6), row(b6),
                Wl2, row(bl2), row(g7), row(b7), Wl3, row(bl3))
